# baseline (device time: 77230 ns/iter reference)
import jax
import jax.numpy as jnp
from jax import lax
from jax.experimental import pallas as pl
from jax.experimental.pallas import tpu as pltpu

N_DEV = 4
B, SQ, DM = 2, 512, 768
HQ, DH = 8, 64
SKV_LOC = 512
WINDOW = 128
BH = B * HQ
QKV_ROWS = BH * SQ


def _fused_attn_allreduce(Qt, Kt, Vt):

    def body(q_ref, k_ref, v_ref, o_out, l_out,
             o_tx, l_tx, o_rx, l_rx, send_sems, recv_sems):
        my = lax.axis_index("i")

        barrier_sem = pltpu.get_barrier_semaphore()
        for d in range(1, N_DEV):
            pl.semaphore_signal(
                barrier_sem, inc=1,
                device_id=((my + d) % N_DEV,),
                device_id_type=pl.DeviceIdType.MESH,
            )
        pl.semaphore_wait(barrier_sem, N_DEV - 1)

        row = lax.broadcasted_iota(jnp.int32, (SQ, SKV_LOC), 0)
        col = lax.broadcasted_iota(jnp.int32, (SQ, SKV_LOC), 1)
        m = jnp.abs(row - (col + my * SKV_LOC)) <= WINDOW
        ones_row = jnp.ones((1, SQ), jnp.bfloat16)

        for bh in range(BH):
            sl = pl.ds(bh * SQ, SQ)
            s = lax.dot_general(
                q_ref[sl, :], k_ref[sl, :],
                (((1,), (1,)), ((), ())),
                preferred_element_type=jnp.float32,
            )
            w = jnp.where(m, jnp.exp(s), 0.0)
            w16 = w.astype(jnp.bfloat16)
            o = lax.dot_general(
                w16, v_ref[sl, :],
                (((1,), (0,)), ((), ())),
                preferred_element_type=jnp.float32,
            )
            o_tx[sl, :] = o.astype(jnp.bfloat16)
            lr = lax.dot_general(
                ones_row, w16,
                (((1,), (1,)), ((), ())),
                preferred_element_type=jnp.float32,
            )
            l_tx[pl.ds(bh, 1), :] = lr.astype(jnp.bfloat16)

        o_rdmas = {}
        l_rdmas = {}
        for d in (2, 1, 3):
            tgt = ((my + d) % N_DEV,)
            o_rdmas[d] = pltpu.make_async_remote_copy(
                src_ref=o_tx, dst_ref=o_rx.at[d - 1],
                send_sem=send_sems.at[d - 1], recv_sem=recv_sems.at[d - 1],
                device_id=tgt, device_id_type=pl.DeviceIdType.MESH,
            )
            o_rdmas[d].start()
            l_rdmas[d] = pltpu.make_async_remote_copy(
                src_ref=l_tx, dst_ref=l_rx.at[d - 1],
                send_sem=send_sems.at[2 + d], recv_sem=recv_sems.at[2 + d],
                device_id=tgt, device_id_type=pl.DeviceIdType.MESH,
            )
            l_rdmas[d].start()

        o_out[...] = o_tx[...].astype(jnp.float32)
        l_out[...] = l_tx[...].astype(jnp.float32)

        for d in (1, 3, 2):
            o_rdmas[d].wait_recv()
            o_out[...] += o_rx[d - 1].astype(jnp.float32)
        for d in (1, 3, 2):
            l_rdmas[d].wait_recv()
            l_out[...] += l_rx[d - 1].astype(jnp.float32)
        for d in (1, 2, 3):
            o_rdmas[d].wait_send()
            l_rdmas[d].wait_send()

    return pl.pallas_call(
        body,
        out_shape=(
            jax.ShapeDtypeStruct((QKV_ROWS, DH), jnp.float32),
            jax.ShapeDtypeStruct((BH, SQ), jnp.float32),
        ),
        in_specs=[pl.BlockSpec(memory_space=pltpu.VMEM)] * 3,
        out_specs=(
            pl.BlockSpec(memory_space=pltpu.VMEM),
            pl.BlockSpec(memory_space=pltpu.VMEM),
        ),
        scratch_shapes=[
            pltpu.VMEM((QKV_ROWS, DH), jnp.bfloat16),
            pltpu.VMEM((BH, SQ), jnp.bfloat16),
            pltpu.VMEM((N_DEV - 1, QKV_ROWS, DH), jnp.bfloat16),
            pltpu.VMEM((N_DEV - 1, BH, SQ), jnp.bfloat16),
            pltpu.SemaphoreType.DMA((6,)),
            pltpu.SemaphoreType.DMA((6,)),
        ],
        compiler_params=pltpu.CompilerParams(collective_id=0),
    )(Qt, Kt, Vt)


def kernel(x, Wq, K_ext, V_ext, Wo):
    q2d = jnp.dot(
        x.reshape(B * SQ, DM).astype(jnp.bfloat16),
        Wq.astype(jnp.bfloat16),
        preferred_element_type=jnp.float32,
    ) * 0.125
    Qt = (
        q2d.reshape(B, SQ, HQ, DH)
        .transpose(0, 2, 1, 3)
        .reshape(QKV_ROWS, DH)
        .astype(jnp.bfloat16)
    )
    Kt = K_ext.astype(jnp.bfloat16).transpose(0, 2, 1, 3).reshape(QKV_ROWS, DH)
    Vt = V_ext.astype(jnp.bfloat16).transpose(0, 2, 1, 3).reshape(QKV_ROWS, DH)

    o_sum, l_sum = _fused_attn_allreduce(Qt, Kt, Vt)

    o4 = o_sum.reshape(B, HQ, SQ, DH)
    l4 = l_sum.reshape(B, HQ, SQ)
    ctx = (o4 / l4[..., None]).transpose(0, 2, 1, 3).reshape(B * SQ, HQ * DH)

    return jnp.dot(
        ctx.astype(jnp.bfloat16),
        Wo.astype(jnp.bfloat16),
        preferred_element_type=jnp.float32,
    ).reshape(B, SQ, DM)
